# baseline (speedup 1.0000x reference)
import functools

import jax
import jax.numpy as jnp
from jax import lax
from jax.experimental import pallas as pl
from jax.experimental.pallas import tpu as pltpu
from jax.experimental.pallas import tpu_sc as plsc

BATCH = 16384
NUM_ROWS = 1000000
EMBED_DIM = 10
NUM_CORES = 2
NUM_SUBCORES = 16
LANES = 16
NUM_WORKERS = NUM_CORES * NUM_SUBCORES
B_PER_W = BATCH // NUM_WORKERS
N_GROUPS = B_PER_W // LANES


def _rsqrt(x):
    i = lax.bitcast_convert_type(x, jnp.int32)
    i = jnp.int32(0x5F3759DF) - lax.shift_right_logical(i, 1)
    y = lax.bitcast_convert_type(i, jnp.float32)
    for _ in range(3):
        y = y * (jnp.float32(1.5) - jnp.float32(0.5) * x * y * y)
    return y


def _body(o_id_hbm, e_id_hbm, mentors_hbm, mentees_hbm, out_hbm,
          oid_v, eid_v, oidx_v, eidx_v, ocols_v, ecols_v, out_v,
          sem_o, sem_e):
    wid = lax.axis_index("s") * NUM_CORES + lax.axis_index("c")
    base = wid * B_PER_W

    pltpu.sync_copy(o_id_hbm.at[pl.ds(base, B_PER_W)], oid_v)
    pltpu.sync_copy(e_id_hbm.at[pl.ds(base, B_PER_W)], eid_v)

    def idx_group(g, carry):
        sl = pl.ds(g * LANES, LANES)
        o = oid_v[sl]
        e = eid_v[sl]
        for d in range(EMBED_DIM):
            off = jnp.int32(d * NUM_ROWS)
            oidx_v[d, sl] = o + off
            eidx_v[d, sl] = e + off
        return carry

    lax.fori_loop(0, N_GROUPS, idx_group, 0)

    copies = []
    for d in range(EMBED_DIM):
        copies.append(pltpu.async_copy(
            mentors_hbm.at[oidx_v.at[d]], ocols_v.at[d], sem_o))
        copies.append(pltpu.async_copy(
            mentees_hbm.at[eidx_v.at[d]], ecols_v.at[d], sem_e))
    for cp in copies:
        cp.wait()

    def group(g, carry):
        sl = pl.ds(g * LANES, LANES)
        dot = jnp.zeros((LANES,), jnp.float32)
        o2 = jnp.zeros((LANES,), jnp.float32)
        e2 = jnp.zeros((LANES,), jnp.float32)
        for d in range(EMBED_DIM):
            o = ocols_v[d, sl]
            e = ecols_v[d, sl]
            dot = dot + o * e
            o2 = o2 + o * o
            e2 = e2 + e * e
        out_v[sl] = dot * _rsqrt(o2 * e2)
        return carry

    lax.fori_loop(0, N_GROUPS, group, 0)

    pltpu.sync_copy(out_v, out_hbm.at[pl.ds(base, B_PER_W)])


@jax.jit
def _cosine(o_id, e_id, mentors, mentees):
    mesh = plsc.VectorSubcoreMesh(core_axis_name="c", subcore_axis_name="s")
    f = functools.partial(
        pl.kernel, mesh=mesh,
        compiler_params=pltpu.CompilerParams(
            needs_layout_passes=False, use_tc_tiling_on_sc=False),
        out_type=jax.ShapeDtypeStruct((BATCH,), jnp.float32),
        scratch_types=[
            pltpu.VMEM((B_PER_W,), jnp.int32),
            pltpu.VMEM((B_PER_W,), jnp.int32),
            pltpu.VMEM((EMBED_DIM, B_PER_W), jnp.int32),
            pltpu.VMEM((EMBED_DIM, B_PER_W), jnp.int32),
            pltpu.VMEM((EMBED_DIM, B_PER_W), jnp.float32),
            pltpu.VMEM((EMBED_DIM, B_PER_W), jnp.float32),
            pltpu.VMEM((B_PER_W,), jnp.float32),
            pltpu.SemaphoreType.DMA,
            pltpu.SemaphoreType.DMA,
        ],
    )(_body)
    mentors_flat = mentors.T.reshape(NUM_ROWS * EMBED_DIM)
    mentees_flat = mentees.T.reshape(NUM_ROWS * EMBED_DIM)
    return f(o_id, e_id, mentors_flat, mentees_flat)


def kernel(o_id, e_id, mentors, mentees):
    return _cosine(o_id.astype(jnp.int32), e_id.astype(jnp.int32),
                   mentors, mentees)


# speedup vs baseline: 1.5089x; 1.5089x over previous
import functools

import jax
import jax.numpy as jnp
from jax import lax
from jax.experimental import pallas as pl
from jax.experimental.pallas import tpu as pltpu
from jax.experimental.pallas import tpu_sc as plsc

BATCH = 16384
NUM_ROWS = 1000000
EMBED_DIM = 10
NUM_CORES = 2
NUM_SUBCORES = 16
LANES = 16
NUM_WORKERS = NUM_CORES * NUM_SUBCORES
B_PER_W = BATCH // NUM_WORKERS
N_GROUPS = B_PER_W // LANES


def _rsqrt(x):
    i = lax.bitcast_convert_type(x, jnp.int32)
    i = jnp.int32(0x5F3759DF) - lax.shift_right_logical(i, 1)
    y = lax.bitcast_convert_type(i, jnp.float32)
    for _ in range(3):
        y = y * (jnp.float32(1.5) - jnp.float32(0.5) * x * y * y)
    return y


def _body(o_id_hbm, e_id_hbm, mentors_hbm, mentees_hbm, out_hbm,
          oid_v, eid_v, oidx_v, eidx_v, ocols_v, ecols_v, out_v,
          sem_o, sem_e):
    wid = lax.axis_index("s") * NUM_CORES + lax.axis_index("c")
    base = wid * B_PER_W

    pltpu.sync_copy(o_id_hbm.at[pl.ds(base, B_PER_W)], oid_v)
    pltpu.sync_copy(e_id_hbm.at[pl.ds(base, B_PER_W)], eid_v)

    def idx_group(g, carry):
        sl = pl.ds(g * LANES, LANES)
        o = oid_v[sl]
        e = eid_v[sl]
        for d in range(EMBED_DIM):
            off = jnp.int32(d * NUM_ROWS)
            oidx_v[d, sl] = o + off
            eidx_v[d, sl] = e + off
        return carry

    lax.fori_loop(0, N_GROUPS, idx_group, 0)

    copies = []
    for d in range(EMBED_DIM):
        copies.append(pltpu.async_copy(
            mentors_hbm.at[oidx_v.at[d]], ocols_v.at[d], sem_o))
        copies.append(pltpu.async_copy(
            mentees_hbm.at[eidx_v.at[d]], ecols_v.at[d], sem_e))
    for cp in copies:
        cp.wait()

    def group(g, carry):
        sl = pl.ds(g * LANES, LANES)
        dot = jnp.zeros((LANES,), jnp.float32)
        o2 = jnp.zeros((LANES,), jnp.float32)
        e2 = jnp.zeros((LANES,), jnp.float32)
        for d in range(EMBED_DIM):
            o = ocols_v[d, sl]
            e = ecols_v[d, sl]
            dot = dot + o * e
            o2 = o2 + o * o
            e2 = e2 + e * e
        out_v[sl] = dot * _rsqrt(o2 * e2)
        return carry

    lax.fori_loop(0, N_GROUPS, group, 0)

    pltpu.sync_copy(out_v, out_hbm.at[pl.ds(base, B_PER_W)])


@jax.jit
def _cosine(o_id, e_id, mentors, mentees):
    mesh = plsc.VectorSubcoreMesh(core_axis_name="c", subcore_axis_name="s")
    f = functools.partial(
        pl.kernel, mesh=mesh,
        compiler_params=pltpu.CompilerParams(
            needs_layout_passes=False, use_tc_tiling_on_sc=False),
        out_type=jax.ShapeDtypeStruct((BATCH,), jnp.float32),
        scratch_types=[
            pltpu.VMEM((B_PER_W,), jnp.int32),
            pltpu.VMEM((B_PER_W,), jnp.int32),
            pltpu.VMEM((EMBED_DIM, B_PER_W), jnp.int32),
            pltpu.VMEM((EMBED_DIM, B_PER_W), jnp.int32),
            pltpu.VMEM((EMBED_DIM, B_PER_W), jnp.float32),
            pltpu.VMEM((EMBED_DIM, B_PER_W), jnp.float32),
            pltpu.VMEM((B_PER_W,), jnp.float32),
            pltpu.SemaphoreType.DMA,
            pltpu.SemaphoreType.DMA,
        ],
    )(_body)
    mentors_flat = jnp.concatenate([mentors[:, d] for d in range(EMBED_DIM)])
    mentees_flat = jnp.concatenate([mentees[:, d] for d in range(EMBED_DIM)])
    return f(o_id, e_id, mentors_flat, mentees_flat)


def kernel(o_id, e_id, mentors, mentees):
    return _cosine(o_id.astype(jnp.int32), e_id.astype(jnp.int32),
                   mentors, mentees)
